# Initial kernel scaffold; baseline (speedup 1.0000x reference)
import jax
import jax.numpy as jnp
from jax.experimental import pallas as pl


def kernel(x_parent, x_child1, x_child2, x_final_descendants, edge_index_parent, edge_index_child1, edge_index_child2, edge_index_final_descendants, W1, b1, W2, b2, W3, b3, We, be):
    raise NotImplementedError("write your pallas kernel here")


# speedup vs baseline: 34.3120x; 34.3120x over previous
import functools

import jax
import jax.numpy as jnp
from jax import lax
from jax.experimental import pallas as pl
from jax.experimental.pallas import tpu as pltpu
from jax.experimental.pallas import tpu_sc as plsc

N = 10000
F16 = 16
CP = 48

NW = 32
CHUNK = 128
NCH = 80
EPAD = NW * NCH * CHUNK
ACC_N = 10240
SROWS = ACC_N // 16
BN = 1000

_mesh = plsc.VectorSubcoreMesh(core_axis_name="c", subcore_axis_name="s")
_sc_params = pltpu.CompilerParams(use_tc_tiling_on_sc=False)


def _zero_fill(zeros_v, nrows):
    z = jnp.zeros((16,), jnp.float32)

    @pl.loop(0, nrows)
    def _(i):
        zeros_v[i] = z


@functools.partial(
    pl.kernel,
    out_type=jax.ShapeDtypeStruct((2, 4, ACC_N, 16), jnp.float32),
    mesh=_mesh,
    compiler_params=_sc_params,
    scratch_types=[
        [pltpu.VMEM_SHARED((ACC_N, 16), jnp.float32) for _ in range(4)],
        pltpu.VMEM((NCH, CHUNK), jnp.int32),
        pltpu.VMEM((CHUNK, 16), jnp.float32),
        pltpu.VMEM((SROWS, 16), jnp.float32),
        [pltpu.SemaphoreType.DMA for _ in range(4)],
    ],
)
def _deg_kernel(d0, d1, d2, d3, out, accs, idx_v, ones_v, zeros_v, sems):
    cid = lax.axis_index("c")
    sid = lax.axis_index("s")
    wid = sid * 2 + cid
    row0 = sid * SROWS

    one = jnp.ones((16,), jnp.float32)

    @pl.loop(0, CHUNK)
    def _(i):
        ones_v[i] = one

    _zero_fill(zeros_v, SROWS)
    for acc in accs:
        pltpu.sync_copy(zeros_v, acc.at[pl.ds(row0, SROWS)])
    plsc.subcore_barrier()

    for dref, acc in zip((d0, d1, d2, d3), accs):
        pltpu.sync_copy(dref.at[wid], idx_v)

        @pl.loop(0, NCH, step=4)
        def _(j):
            for b in range(4):
                jj = j + b

                @pl.when(jj >= 4)
                def _():
                    pltpu.make_async_copy(
                        ones_v, acc.at[idx_v.at[jj - 4]], sems[b]).wait()

                pltpu.async_copy(ones_v, acc.at[idx_v.at[jj]], sems[b],
                                 add=True)

        for b in range(4):
            pltpu.make_async_copy(
                ones_v, acc.at[idx_v.at[NCH - 4 + b]], sems[b]).wait()

    plsc.subcore_barrier()
    for k, acc in enumerate(accs):
        pltpu.sync_copy(acc.at[pl.ds(row0, SROWS)],
                        out.at[cid, k, pl.ds(row0, SROWS)])


def _make_msg_kernel(nsets, width):
    @functools.partial(
        pl.kernel,
        out_type=jax.ShapeDtypeStruct((2, nsets, ACC_N, width), jnp.float32),
        mesh=_mesh,
        compiler_params=_sc_params,
        scratch_types=[
            [pltpu.VMEM_SHARED((ACC_N, width), jnp.float32)
             for _ in range(nsets)],
            pltpu.VMEM((NCH, CHUNK), jnp.int32),
            pltpu.VMEM((NCH, CHUNK), jnp.int32),
            [pltpu.VMEM((CHUNK, width), jnp.float32) for _ in range(4)],
            pltpu.VMEM((SROWS, width), jnp.float32),
            [pltpu.SemaphoreType.DMA for _ in range(4)],
            [pltpu.SemaphoreType.DMA for _ in range(4)],
        ],
    )
    def msg_kernel(*args):
        gs = args[:nsets]
        srefs = args[nsets:2 * nsets]
        drefs = args[2 * nsets:3 * nsets]
        out = args[3 * nsets]
        accs, sidx_v, didx_v, bufs, zeros_v, gsems, ssems = args[3 * nsets + 1:]

        cid = lax.axis_index("c")
        sid = lax.axis_index("s")
        wid = sid * 2 + cid
        row0 = sid * SROWS

        for w0 in range(0, width, 16):
            z = jnp.zeros((16,), jnp.float32)

            @pl.loop(0, SROWS)
            def _(i):
                zeros_v[i, pl.ds(w0, 16)] = z

        for acc in accs:
            pltpu.sync_copy(zeros_v, acc.at[pl.ds(row0, SROWS)])
        plsc.subcore_barrier()

        for g, sref, dref, acc in zip(gs, srefs, drefs, accs):
            pltpu.sync_copy(sref.at[wid], sidx_v)
            pltpu.sync_copy(dref.at[wid], didx_v)

            @pl.loop(0, NCH + 4, step=4)
            def _(j):
                for b in range(4):
                    jj = j + b

                    @pl.when(jj < NCH)
                    def _():
                        @pl.when(jj >= 4)
                        def _():
                            pltpu.make_async_copy(
                                bufs[b], acc.at[didx_v.at[jj - 4]],
                                ssems[b]).wait()

                        pltpu.async_copy(g.at[sidx_v.at[jj]], bufs[b],
                                         gsems[b])

                    jk = jj - 2
                    b2 = (b - 2) % 4

                    @pl.when((jk >= 0) & (jk < NCH))
                    def _():
                        pltpu.make_async_copy(
                            g.at[sidx_v.at[jk]], bufs[b2], gsems[b2]).wait()
                        pltpu.async_copy(bufs[b2], acc.at[didx_v.at[jk]],
                                         ssems[b2], add=True)

            for b in range(4):
                pltpu.make_async_copy(
                    bufs[b], acc.at[didx_v.at[NCH - 4 + b]], ssems[b]).wait()

        plsc.subcore_barrier()
        for k, acc in enumerate(accs):
            pltpu.sync_copy(acc.at[pl.ds(row0, SROWS)],
                            out.at[cid, k, pl.ds(row0, SROWS)])

    return msg_kernel


_msg4 = _make_msg_kernel(4, F16)
_msg1 = _make_msg_kernel(1, CP)


def _tc1_body(x_ref, w_ref, dacc_ref, g0_ref, g1_ref, g2_ref, g3_ref):
    outs = (g0_ref, g1_ref, g2_ref, g3_ref)
    for k in range(4):
        deg = 1.0 + dacc_ref[0, k] + dacc_ref[1, k]
        dinv = lax.rsqrt(deg)
        h = jnp.dot(x_ref[k], w_ref[k], preferred_element_type=jnp.float32)
        outs[k][...] = h * dinv


def _tc2_body(g0_ref, g1_ref, g2_ref, g3_ref, macc_ref, dacc_ref, bs_ref,
              we_ref, g5_ref):
    gr = (g0_ref, g1_ref, g2_ref, g3_ref)
    outs = []
    for k in range(4):
        deg = 1.0 + dacc_ref[0, k] + dacc_ref[1, k]
        dinv = lax.rsqrt(deg)
        acc = macc_ref[0, k] + macc_ref[1, k]
        outs.append(dinv * (acc + gr[k][...]) + bs_ref[k])
    xp = outs[0]
    x = jnp.maximum(xp, 0.0)
    xc1 = outs[1] + x + xp
    x = jnp.maximum(xc1, 0.0)
    xc2 = outs[2] + x + xc1
    x = jnp.maximum(xc2, 0.0)
    xf = outs[3] + x + xc1 + xc2
    x4 = jnp.maximum(xf, 0.0)
    h5 = jnp.dot(x4, we_ref[...], preferred_element_type=jnp.float32)
    deg5 = 1.0 + dacc_ref[0, 3] + dacc_ref[1, 3]
    dinv5 = lax.rsqrt(deg5)[:, 0:1]
    g5_ref[...] = h5 * dinv5


def _tc3_body(g5_ref, m5_ref, dacc_ref, be_ref, o_ref):
    deg = 1.0 + dacc_ref[0, 0] + dacc_ref[1, 0]
    dinv = lax.rsqrt(deg)[:, 0:1]
    out = dinv * (m5_ref[0, 0] + m5_ref[1, 0] + g5_ref[...]) + be_ref[...]
    logits = out[:, :40]
    mx = jnp.max(logits, axis=1, keepdims=True)
    lse = mx + jnp.log(jnp.sum(jnp.exp(logits - mx), axis=1, keepdims=True))
    o_ref[...] = logits - lse


def _pad_cols(a, w):
    return jnp.pad(a, ((0, 0), (0, w - a.shape[1])))


def _prep_edges(ei):
    e = ei.shape[1]
    pad = EPAD - e
    src = jnp.concatenate([ei[0], jnp.zeros((pad,), jnp.int32)])
    dst = jnp.concatenate([ei[1], jnp.full((pad,), N, jnp.int32)])
    return src.reshape(NW, NCH, CHUNK), dst.reshape(NW, NCH, CHUNK)


def kernel(x_parent, x_child1, x_child2, x_final_descendants,
           edge_index_parent, edge_index_child1, edge_index_child2,
           edge_index_final_descendants,
           W1, b1, W2, b2, W3, b3, We, be):
    f = jnp.float32
    FP = 130
    xs = jnp.stack([_pad_cols(x_parent.astype(f), FP),
                    _pad_cols(x_child1.astype(f), FP),
                    _pad_cols(x_child2.astype(f), FP),
                    _pad_cols(x_final_descendants.astype(f), FP)])
    Ws = jnp.stack([jnp.pad(W1, ((0, FP - W1.shape[0]), (0, 0))),
                    jnp.pad(W2, ((0, FP - W2.shape[0]), (0, 0))),
                    jnp.pad(W3, ((0, FP - W3.shape[0]), (0, 0))),
                    jnp.pad(W2, ((0, FP - W2.shape[0]), (0, 0)))])
    bs = jnp.stack([b1, b2, b3, b2])
    We_p = jnp.pad(We, ((0, 0), (0, CP - We.shape[1])))
    be_p = jnp.pad(be, ((0, CP - be.shape[0]),))

    s0, d0 = _prep_edges(edge_index_parent)
    s1, d1 = _prep_edges(edge_index_child1)
    s2, d2 = _prep_edges(edge_index_child2)
    s3, d3 = _prep_edges(edge_index_final_descendants)

    dacc = _deg_kernel(d0, d1, d2, d3)

    g0, g1, g2, g3 = pl.pallas_call(
        _tc1_body,
        grid=(N // BN,),
        in_specs=[
            pl.BlockSpec((4, BN, FP), lambda i: (0, i, 0)),
            pl.BlockSpec((4, FP, F16), lambda i: (0, 0, 0)),
            pl.BlockSpec((2, 4, BN, F16), lambda i: (0, 0, i, 0)),
        ],
        out_specs=[pl.BlockSpec((BN, F16), lambda i: (i, 0))] * 4,
        out_shape=[jax.ShapeDtypeStruct((N, F16), f)] * 4,
    )(xs, Ws, dacc)

    macc = _msg4(g0, g1, g2, g3, s0, s1, s2, s3, d0, d1, d2, d3)

    g5 = pl.pallas_call(
        _tc2_body,
        grid=(N // BN,),
        in_specs=[pl.BlockSpec((BN, F16), lambda i: (i, 0))] * 4 + [
            pl.BlockSpec((2, 4, BN, F16), lambda i: (0, 0, i, 0)),
            pl.BlockSpec((2, 4, BN, F16), lambda i: (0, 0, i, 0)),
            pl.BlockSpec((4, F16), lambda i: (0, 0)),
            pl.BlockSpec((F16, CP), lambda i: (0, 0)),
        ],
        out_specs=pl.BlockSpec((BN, CP), lambda i: (i, 0)),
        out_shape=jax.ShapeDtypeStruct((N, CP), f),
    )(g0, g1, g2, g3, macc, dacc, bs, We_p)

    m5 = _msg1(g5, s3, d3)

    out = pl.pallas_call(
        _tc3_body,
        grid=(N // BN,),
        in_specs=[
            pl.BlockSpec((BN, CP), lambda i: (i, 0)),
            pl.BlockSpec((2, 1, BN, CP), lambda i: (0, 0, i, 0)),
            pl.BlockSpec((2, 1, BN, F16), lambda i: (0, 3, i, 0)),
            pl.BlockSpec((CP,), lambda i: (0,)),
        ],
        out_specs=pl.BlockSpec((BN, 40), lambda i: (i, 0)),
        out_shape=jax.ShapeDtypeStruct((N, 40), f),
    )(g5, m5, dacc, be_p)

    return out


# speedup vs baseline: 42.2872x; 1.2324x over previous
import functools

import jax
import jax.numpy as jnp
from jax import lax
from jax.experimental import pallas as pl
from jax.experimental.pallas import tpu as pltpu
from jax.experimental.pallas import tpu_sc as plsc

N = 10000
F16 = 16
DEPTH = 8
LEAD = 4

NW = 32
CHUNK = 128
NCH = 80
EPAD = NW * NCH * CHUNK
ACC_N = 10240
SROWS = ACC_N // 16
BN = 1000

_mesh = plsc.VectorSubcoreMesh(core_axis_name="c", subcore_axis_name="s")
_sc_params = pltpu.CompilerParams(use_tc_tiling_on_sc=False)


def _zero_fill(zeros_v, nrows):
    z = jnp.zeros((16,), jnp.float32)

    @pl.loop(0, nrows)
    def _(i):
        zeros_v[i] = z


@functools.partial(
    pl.kernel,
    out_type=jax.ShapeDtypeStruct((2, 4, ACC_N, 16), jnp.float32),
    mesh=_mesh,
    compiler_params=_sc_params,
    scratch_types=[
        [pltpu.VMEM_SHARED((ACC_N, 16), jnp.float32) for _ in range(4)],
        pltpu.VMEM((NCH, CHUNK), jnp.int32),
        pltpu.VMEM((CHUNK, 16), jnp.float32),
        pltpu.VMEM((SROWS, 16), jnp.float32),
        [pltpu.SemaphoreType.DMA for _ in range(DEPTH)],
    ],
)
def _deg_kernel(d0, d1, d2, d3, out, accs, idx_v, ones_v, zeros_v, sems):
    cid = lax.axis_index("c")
    sid = lax.axis_index("s")
    wid = sid * 2 + cid
    row0 = sid * SROWS

    one = jnp.ones((16,), jnp.float32)

    @pl.loop(0, CHUNK)
    def _(i):
        ones_v[i] = one

    _zero_fill(zeros_v, SROWS)
    for acc in accs:
        pltpu.sync_copy(zeros_v, acc.at[pl.ds(row0, SROWS)])
    plsc.subcore_barrier()

    for dref, acc in zip((d0, d1, d2, d3), accs):
        pltpu.sync_copy(dref.at[wid], idx_v)

        @pl.loop(0, NCH, step=DEPTH)
        def _(j):
            for b in range(DEPTH):
                jj = j + b

                @pl.when(jj >= DEPTH)
                def _():
                    pltpu.make_async_copy(
                        ones_v, acc.at[idx_v.at[jj - DEPTH]], sems[b]).wait()

                pltpu.async_copy(ones_v, acc.at[idx_v.at[jj]], sems[b],
                                 add=True)

        for b in range(DEPTH):
            pltpu.make_async_copy(
                ones_v, acc.at[idx_v.at[NCH - DEPTH + b]], sems[b]).wait()

    plsc.subcore_barrier()
    for k, acc in enumerate(accs):
        pltpu.sync_copy(acc.at[pl.ds(row0, SROWS)],
                        out.at[cid, k, pl.ds(row0, SROWS)])


def _make_msg_kernel(nsets, width):
    @functools.partial(
        pl.kernel,
        out_type=jax.ShapeDtypeStruct((2, nsets, ACC_N, width), jnp.float32),
        mesh=_mesh,
        compiler_params=_sc_params,
        scratch_types=[
            [pltpu.VMEM_SHARED((ACC_N, width), jnp.float32)
             for _ in range(nsets)],
            pltpu.VMEM((NCH, CHUNK), jnp.int32),
            pltpu.VMEM((NCH, CHUNK), jnp.int32),
            [pltpu.VMEM((CHUNK, width), jnp.float32) for _ in range(DEPTH)],
            pltpu.VMEM((SROWS, width), jnp.float32),
            [pltpu.SemaphoreType.DMA for _ in range(DEPTH)],
            [pltpu.SemaphoreType.DMA for _ in range(DEPTH)],
        ],
    )
    def msg_kernel(*args):
        gs = args[:nsets]
        srefs = args[nsets:2 * nsets]
        drefs = args[2 * nsets:3 * nsets]
        out = args[3 * nsets]
        accs, sidx_v, didx_v, bufs, zeros_v, gsems, ssems = args[3 * nsets + 1:]

        cid = lax.axis_index("c")
        sid = lax.axis_index("s")
        wid = sid * 2 + cid
        row0 = sid * SROWS

        for w0 in range(0, width, 16):
            z = jnp.zeros((16,), jnp.float32)

            @pl.loop(0, SROWS)
            def _(i):
                zeros_v[i, pl.ds(w0, 16)] = z

        for acc in accs:
            pltpu.sync_copy(zeros_v, acc.at[pl.ds(row0, SROWS)])
        plsc.subcore_barrier()

        for g, sref, dref, acc in zip(gs, srefs, drefs, accs):
            pltpu.sync_copy(sref.at[wid], sidx_v)
            pltpu.sync_copy(dref.at[wid], didx_v)

            @pl.loop(0, NCH + DEPTH, step=DEPTH)
            def _(j):
                for b in range(DEPTH):
                    jj = j + b

                    @pl.when(jj < NCH)
                    def _():
                        @pl.when(jj >= DEPTH)
                        def _():
                            pltpu.make_async_copy(
                                bufs[b], acc.at[didx_v.at[jj - DEPTH]],
                                ssems[b]).wait()

                        pltpu.async_copy(g.at[sidx_v.at[jj]], bufs[b],
                                         gsems[b])

                    jk = jj - LEAD
                    b2 = (b - LEAD) % DEPTH

                    @pl.when((jk >= 0) & (jk < NCH))
                    def _():
                        pltpu.make_async_copy(
                            g.at[sidx_v.at[jk]], bufs[b2], gsems[b2]).wait()
                        pltpu.async_copy(bufs[b2], acc.at[didx_v.at[jk]],
                                         ssems[b2], add=True)

            for b in range(DEPTH):
                pltpu.make_async_copy(
                    bufs[b], acc.at[didx_v.at[NCH - DEPTH + b]],
                    ssems[b]).wait()

        plsc.subcore_barrier()
        for k, acc in enumerate(accs):
            pltpu.sync_copy(acc.at[pl.ds(row0, SROWS)],
                            out.at[cid, k, pl.ds(row0, SROWS)])

    return msg_kernel


_msg4 = _make_msg_kernel(4, F16)
_msg1 = _make_msg_kernel(1, F16)


def _tc1_body(x_ref, w_ref, dacc_ref, g0_ref, g1_ref, g2_ref, g3_ref):
    outs = (g0_ref, g1_ref, g2_ref, g3_ref)
    for k in range(4):
        deg = 1.0 + dacc_ref[0, k] + dacc_ref[1, k]
        dinv = lax.rsqrt(deg)
        h = jnp.dot(x_ref[k], w_ref[k], preferred_element_type=jnp.float32)
        outs[k][...] = h * dinv


def _tc2_body(g0_ref, g1_ref, g2_ref, g3_ref, macc_ref, dacc_ref, bs_ref,
              u_ref):
    gr = (g0_ref, g1_ref, g2_ref, g3_ref)
    outs = []
    for k in range(4):
        deg = 1.0 + dacc_ref[0, k] + dacc_ref[1, k]
        dinv = lax.rsqrt(deg)
        acc = macc_ref[0, k] + macc_ref[1, k]
        outs.append(dinv * (acc + gr[k][...]) + bs_ref[k])
    xp = outs[0]
    x = jnp.maximum(xp, 0.0)
    xc1 = outs[1] + x + xp
    x = jnp.maximum(xc1, 0.0)
    xc2 = outs[2] + x + xc1
    x = jnp.maximum(xc2, 0.0)
    xf = outs[3] + x + xc1 + xc2
    x4 = jnp.maximum(xf, 0.0)
    deg5 = 1.0 + dacc_ref[0, 3] + dacc_ref[1, 3]
    dinv5 = lax.rsqrt(deg5)
    u_ref[...] = x4 * dinv5


def _tc3_body(u_ref, m5_ref, dacc_ref, we_ref, be_ref, o_ref):
    deg = 1.0 + dacc_ref[0, 0] + dacc_ref[1, 0]
    dinv = lax.rsqrt(deg)[:, 0:1]
    s = m5_ref[0, 0] + m5_ref[1, 0] + u_ref[...]
    h = jnp.dot(s, we_ref[...], preferred_element_type=jnp.float32)
    logits = dinv * h + be_ref[...]
    mx = jnp.max(logits, axis=1, keepdims=True)
    lse = mx + jnp.log(jnp.sum(jnp.exp(logits - mx), axis=1, keepdims=True))
    o_ref[...] = logits - lse


def _pad_cols(a, w):
    return jnp.pad(a, ((0, 0), (0, w - a.shape[1])))


def _prep_edges(ei):
    e = ei.shape[1]
    pad = EPAD - e
    src = jnp.concatenate([ei[0], jnp.zeros((pad,), jnp.int32)])
    dst = jnp.concatenate([ei[1], jnp.full((pad,), N, jnp.int32)])
    return src.reshape(NW, NCH, CHUNK), dst.reshape(NW, NCH, CHUNK)


def kernel(x_parent, x_child1, x_child2, x_final_descendants,
           edge_index_parent, edge_index_child1, edge_index_child2,
           edge_index_final_descendants,
           W1, b1, W2, b2, W3, b3, We, be):
    f = jnp.float32
    FP = 130
    xs = jnp.stack([_pad_cols(x_parent.astype(f), FP),
                    _pad_cols(x_child1.astype(f), FP),
                    _pad_cols(x_child2.astype(f), FP),
                    _pad_cols(x_final_descendants.astype(f), FP)])
    Ws = jnp.stack([jnp.pad(W1, ((0, FP - W1.shape[0]), (0, 0))),
                    jnp.pad(W2, ((0, FP - W2.shape[0]), (0, 0))),
                    jnp.pad(W3, ((0, FP - W3.shape[0]), (0, 0))),
                    jnp.pad(W2, ((0, FP - W2.shape[0]), (0, 0)))])
    bs = jnp.stack([b1, b2, b3, b2])

    s0, d0 = _prep_edges(edge_index_parent)
    s1, d1 = _prep_edges(edge_index_child1)
    s2, d2 = _prep_edges(edge_index_child2)
    s3, d3 = _prep_edges(edge_index_final_descendants)

    dacc = _deg_kernel(d0, d1, d2, d3)

    g0, g1, g2, g3 = pl.pallas_call(
        _tc1_body,
        grid=(N // BN,),
        in_specs=[
            pl.BlockSpec((4, BN, FP), lambda i: (0, i, 0)),
            pl.BlockSpec((4, FP, F16), lambda i: (0, 0, 0)),
            pl.BlockSpec((2, 4, BN, F16), lambda i: (0, 0, i, 0)),
        ],
        out_specs=[pl.BlockSpec((BN, F16), lambda i: (i, 0))] * 4,
        out_shape=[jax.ShapeDtypeStruct((N, F16), f)] * 4,
    )(xs, Ws, dacc)

    macc = _msg4(g0, g1, g2, g3, s0, s1, s2, s3, d0, d1, d2, d3)

    u = pl.pallas_call(
        _tc2_body,
        grid=(N // BN,),
        in_specs=[pl.BlockSpec((BN, F16), lambda i: (i, 0))] * 4 + [
            pl.BlockSpec((2, 4, BN, F16), lambda i: (0, 0, i, 0)),
            pl.BlockSpec((2, 4, BN, F16), lambda i: (0, 0, i, 0)),
            pl.BlockSpec((4, F16), lambda i: (0, 0)),
        ],
        out_specs=pl.BlockSpec((BN, F16), lambda i: (i, 0)),
        out_shape=jax.ShapeDtypeStruct((N, F16), f),
    )(g0, g1, g2, g3, macc, dacc, bs)

    m5 = _msg1(u, s3, d3)

    out = pl.pallas_call(
        _tc3_body,
        grid=(N // BN,),
        in_specs=[
            pl.BlockSpec((BN, F16), lambda i: (i, 0)),
            pl.BlockSpec((2, 1, BN, F16), lambda i: (0, 0, i, 0)),
            pl.BlockSpec((2, 1, BN, F16), lambda i: (0, 3, i, 0)),
            pl.BlockSpec((F16, 40), lambda i: (0, 0)),
            pl.BlockSpec((40,), lambda i: (0,)),
        ],
        out_specs=pl.BlockSpec((BN, 40), lambda i: (i, 0)),
        out_shape=jax.ShapeDtypeStruct((N, 40), f),
    )(u, m5, dacc, We, be)

    return out
